# Initial kernel scaffold; baseline (speedup 1.0000x reference)
import jax
import jax.numpy as jnp
from jax.experimental import pallas as pl


def kernel(x, gate_w, gate_b, expert_w, expert_b):
    raise NotImplementedError("write your pallas kernel here")


# speedup vs baseline: 2.3971x; 2.3971x over previous
import functools

import jax
import jax.numpy as jnp
from jax.experimental import pallas as pl
from jax.experimental.pallas import tpu as pltpu

B, S, D, H, E = 2, 2048, 1024, 1024, 8
N = B * S
TM = 256


def _fused_body(x_ref, gw_ref, gb_ref, ew_ref, eb_ref,
                w_out, y1_out, y2_out, i1_s, i2_s):
    e = pl.program_id(1)

    @pl.when(e == 0)
    def _gate():
        x = x_ref[...]
        logits = jnp.dot(x, gw_ref[...], preferred_element_type=jnp.float32)
        logits = logits + gb_ref[...]
        z = logits - jnp.max(logits, axis=1, keepdims=True)
        p = jnp.exp(z)
        p = p / jnp.sum(p, axis=1, keepdims=True)
        iota = jax.lax.broadcasted_iota(jnp.int32, (TM, E), 1)
        w1 = jnp.max(p, axis=1, keepdims=True)
        i1 = jnp.min(jnp.where(p == w1, iota, E), axis=1, keepdims=True)
        p2 = jnp.where(iota == i1, -1.0, p)
        w2 = jnp.max(p2, axis=1, keepdims=True)
        i2 = jnp.min(jnp.where(p2 == w2, iota, E), axis=1, keepdims=True)
        i1_s[...] = i1
        i2_s[...] = i2
        w_out[...] = jnp.concatenate([w1, w2], axis=1)

    y_e = jnp.dot(x_ref[...], ew_ref[0], preferred_element_type=jnp.float32)
    y_e = y_e + eb_ref[0]
    m1 = i1_s[...] == e
    m2 = i2_s[...] == e
    y1_out[...] = jnp.where(m1, y_e, jnp.where(e == 0, 0.0, y1_out[...]))
    y2_out[...] = jnp.where(m2, y_e, jnp.where(e == 0, 0.0, y2_out[...]))


@functools.partial(jax.jit, static_argnames=("interpret",))
def _fused(x, gate_w, gate_b, expert_w, expert_b, interpret=False):
    xf = x.reshape(N, D)
    grid = (N // TM, E)
    w_out, y1, y2 = pl.pallas_call(
        _fused_body,
        grid=grid,
        in_specs=[
            pl.BlockSpec((TM, D), lambda m, e: (m, 0)),
            pl.BlockSpec((D, E), lambda m, e: (0, 0)),
            pl.BlockSpec((1, E), lambda m, e: (0, 0)),
            pl.BlockSpec((1, D, H), lambda m, e: (e, 0, 0)),
            pl.BlockSpec((1, 1, H), lambda m, e: (e, 0, 0)),
        ],
        out_specs=[
            pl.BlockSpec((TM, 2), lambda m, e: (m, 0)),
            pl.BlockSpec((TM, H), lambda m, e: (m, 0)),
            pl.BlockSpec((TM, H), lambda m, e: (m, 0)),
        ],
        out_shape=[
            jax.ShapeDtypeStruct((N, 2), jnp.float32),
            jax.ShapeDtypeStruct((N, H), jnp.float32),
            jax.ShapeDtypeStruct((N, H), jnp.float32),
        ],
        scratch_shapes=[
            pltpu.VMEM((TM, 1), jnp.int32),
            pltpu.VMEM((TM, 1), jnp.int32),
        ],
        interpret=interpret,
    )(xf, gate_w, gate_b.reshape(1, E), expert_w, expert_b.reshape(E, 1, H))
    top2_w = w_out.reshape(B, S, 2)
    top2_y = jnp.stack([y1, y2], axis=1).reshape(B, S, 2, H)
    return top2_w, top2_y


def kernel(x, gate_w, gate_b, expert_w, expert_b):
    return _fused(x, gate_w, gate_b, expert_w, expert_b)


# speedup vs baseline: 3.6291x; 1.5140x over previous
import functools

import jax
import jax.numpy as jnp
from jax import lax
from jax.experimental import pallas as pl
from jax.experimental.pallas import tpu as pltpu
from jax.experimental.pallas import tpu_sc as plsc

B, S, D, H, E = 2, 2048, 1024, 1024, 8
N = B * S
P = 2 * N
TM = 128
NP = P + E * TM
NT = NP // TM

NW = 32
TW = N // NW
CH = 16


def _lane_cumsum(v):
    k = 1
    while k < N:
        sh = jnp.concatenate(
            [jnp.zeros((1, k), jnp.int32), v[:, : N - k]], axis=1)
        v = v + sh
        k *= 2
    return v


def _route_body(x_ref, gw_ref, gb_ref, w_out, pos_out, te_out):
    logits = jnp.dot(x_ref[...], gw_ref[...],
                     preferred_element_type=jnp.float32)
    i8 = (lax.broadcasted_iota(jnp.int32, (E, E), 0)
          == lax.broadcasted_iota(jnp.int32, (E, E), 1)).astype(jnp.float32)
    lt = lax.dot_general(i8, logits, (((1,), (1,)), ((), ())),
                         preferred_element_type=jnp.float32,
                         precision=lax.Precision.HIGHEST)
    lt = lt + gb_ref[...]
    m = jnp.max(lt, axis=0, keepdims=True)
    p = jnp.exp(lt - m)
    p = p / jnp.sum(p, axis=0, keepdims=True)
    si = lax.broadcasted_iota(jnp.int32, (E, N), 0)
    w1 = jnp.max(p, axis=0, keepdims=True)
    e0 = jnp.min(jnp.where(p == w1, si, E), axis=0, keepdims=True)
    p2 = jnp.where(si == e0, -1.0, p)
    w2 = jnp.max(p2, axis=0, keepdims=True)
    e1 = jnp.min(jnp.where(p2 == w2, si, E), axis=0, keepdims=True)
    w_out[0:1, :] = w1
    w_out[1:2, :] = w2

    pos0 = jnp.zeros((1, N), jnp.int32)
    pos1 = jnp.zeros((1, N), jnp.int32)
    po = jnp.int32(0)
    po_list = []
    for e in range(E):
        ind0 = e0 == e
        ind1 = e1 == e
        i0 = ind0.astype(jnp.int32)
        i1 = ind1.astype(jnp.int32)
        sexc = (_lane_cumsum(i0) - i0) + (_lane_cumsum(i1) - i1)
        po_list.append(po)
        pos0 = pos0 + jnp.where(ind0, sexc + po, 0)
        pos1 = pos1 + jnp.where(ind1, sexc + po, 0)
        ce = jnp.sum(i0) + jnp.sum(i1)
        po = po + (ce + TM - 1) // TM * TM
    pos_out[0:1, :] = pos0
    pos_out[1:2, :] = pos1

    tiles = lax.broadcasted_iota(jnp.int32, (1, NT), 1) * TM
    te = jnp.zeros((1, NT), jnp.int32)
    for e in range(1, E):
        te = te + (tiles >= po_list[e]).astype(jnp.int32)
    te_out[...] = te


def _route(xf, gate_w, gate_b, interpret=False):
    return pl.pallas_call(
        _route_body,
        out_shape=[
            jax.ShapeDtypeStruct((2, N), jnp.float32),
            jax.ShapeDtypeStruct((2, N), jnp.int32),
            jax.ShapeDtypeStruct((1, NT), jnp.int32),
        ],
        interpret=interpret,
    )(xf, gate_w, gate_b.reshape(E, 1))


def _gmm_body(te_ref, xs_ref, ew_ref, eb_ref, y_ref):
    y_ref[...] = (jnp.dot(xs_ref[...], ew_ref[0],
                          preferred_element_type=jnp.float32) + eb_ref[0])


def _gmm(te, xs, expert_w, expert_b, interpret=False):
    return pl.pallas_call(
        _gmm_body,
        grid_spec=pltpu.PrefetchScalarGridSpec(
            num_scalar_prefetch=1,
            grid=(NT,),
            in_specs=[
                pl.BlockSpec((TM, D), lambda t, te: (t, 0)),
                pl.BlockSpec((1, D, H), lambda t, te: (te[t], 0, 0)),
                pl.BlockSpec((1, 1, H), lambda t, te: (te[t], 0, 0)),
            ],
            out_specs=pl.BlockSpec((TM, H), lambda t, te: (t, 0)),
        ),
        out_shape=jax.ShapeDtypeStruct((NP, H), jnp.float32),
        interpret=interpret,
    )(te, xs, expert_w, expert_b.reshape(E, 1, H))


@functools.lru_cache(maxsize=None)
def _sc_kernels():
    mesh = plsc.VectorSubcoreMesh(core_axis_name="c", subcore_axis_name="s")

    @functools.partial(
        pl.kernel,
        mesh=mesh,
        out_type=jax.ShapeDtypeStruct((NP, D), jnp.float32),
        scratch_types=[
            pltpu.VMEM((TW,), jnp.int32),
            pltpu.VMEM((TW,), jnp.int32),
            pltpu.VMEM((CH, D), jnp.float32),
            pltpu.SemaphoreType.DMA,
            pltpu.SemaphoreType.DMA,
        ],
    )
    def sc_scatter(x_hbm, pos_hbm, xs_hbm, p0_v, p1_v, xbuf, sem0, sem1):
        wid = lax.axis_index("s") * 2 + lax.axis_index("c")
        tbase = wid * TW
        pltpu.sync_copy(pos_hbm.at[0, pl.ds(tbase, TW)], p0_v)
        pltpu.sync_copy(pos_hbm.at[1, pl.ds(tbase, TW)], p1_v)

        def chunk(c, carry):
            t0 = tbase + c * CH
            pltpu.sync_copy(x_hbm.at[pl.ds(t0, CH)], xbuf)
            idx0 = p0_v[pl.ds(c * CH, CH)]
            idx1 = p1_v[pl.ds(c * CH, CH)]
            cp0 = pltpu.async_copy(xbuf, xs_hbm.at[idx0], sem0)
            cp1 = pltpu.async_copy(xbuf, xs_hbm.at[idx1], sem1)
            cp0.wait()
            cp1.wait()
            return carry

        lax.fori_loop(0, TW // CH, chunk, 0)

    @functools.partial(
        pl.kernel,
        mesh=mesh,
        out_type=jax.ShapeDtypeStruct((P, H), jnp.float32),
        scratch_types=[
            pltpu.VMEM((TW,), jnp.int32),
            pltpu.VMEM((TW,), jnp.int32),
            pltpu.VMEM((CH, H), jnp.float32),
            pltpu.VMEM((CH, H), jnp.float32),
            pltpu.SemaphoreType.DMA,
            pltpu.SemaphoreType.DMA,
        ],
    )
    def sc_gather(ys_hbm, pos_hbm, out_hbm, p0_v, p1_v, buf0, buf1, sem0, sem1):
        wid = lax.axis_index("s") * 2 + lax.axis_index("c")
        tbase = wid * TW
        pltpu.sync_copy(pos_hbm.at[0, pl.ds(tbase, TW)], p0_v)
        pltpu.sync_copy(pos_hbm.at[1, pl.ds(tbase, TW)], p1_v)

        def chunk(c, carry):
            idx0 = p0_v[pl.ds(c * CH, CH)]
            idx1 = p1_v[pl.ds(c * CH, CH)]
            g0 = pltpu.async_copy(ys_hbm.at[idx0], buf0, sem0)
            g1 = pltpu.async_copy(ys_hbm.at[idx1], buf1, sem1)
            t0 = tbase + c * CH
            iot = lax.iota(jnp.int32, CH)
            dest0 = (t0 + iot) * 2
            dest1 = dest0 + 1
            g0.wait()
            s0 = pltpu.async_copy(buf0, out_hbm.at[dest0], sem0)
            g1.wait()
            s1 = pltpu.async_copy(buf1, out_hbm.at[dest1], sem1)
            s0.wait()
            s1.wait()
            return carry

        lax.fori_loop(0, TW // CH, chunk, 0)

    return sc_scatter, sc_gather


@jax.jit
def _moe(x, gate_w, gate_b, expert_w, expert_b):
    xf = x.reshape(N, D)
    sc_scatter, sc_gather = _sc_kernels()
    w2n, pos, te = _route(xf, gate_w, gate_b)
    xs = sc_scatter(xf, pos)
    ys = _gmm(te.reshape(NT), xs, expert_w, expert_b)
    yout = sc_gather(ys, pos)
    top2_w = w2n.T.reshape(B, S, 2)
    top2_y = yout.reshape(B, S, 2, H)
    return top2_w, top2_y


def kernel(x, gate_w, gate_b, expert_w, expert_b):
    return _moe(x, gate_w, gate_b, expert_w, expert_b)
